# Initial kernel scaffold; baseline (speedup 1.0000x reference)
import jax
import jax.numpy as jnp
from jax.experimental import pallas as pl


def kernel(x, input_mask, rotations):
    raise NotImplementedError("write your pallas kernel here")


# speedup vs baseline: 5.1330x; 5.1330x over previous
import functools

import jax
import jax.numpy as jnp
from jax import lax
from jax.experimental import pallas as pl

_N = 2048
_D = 768
_NB = 8
_BLK = 256
_EPS = 1e-5
_NEG = -1e30


def _prep_body(x_ref, mask_ref, rot_ref, xm_ref, oh_ref):
    x = x_ref[...]
    mu = jnp.mean(x, axis=1, keepdims=True)
    var = jnp.mean((x - mu) ** 2, axis=1, keepdims=True)
    xn = (x - mu) * lax.rsqrt(var + _EPS)
    s = jnp.dot(xn, rot_ref[...], preferred_element_type=jnp.float32)
    smax = jnp.max(s, axis=1, keepdims=True)
    idx = lax.broadcasted_iota(jnp.int32, s.shape, 1)
    cand = jnp.where(s == smax, idx, _NB)
    first = jnp.min(cand, axis=1, keepdims=True)
    oh_ref[...] = (idx == first).astype(jnp.float32)
    xm_ref[...] = xn * mask_ref[...]


def _attn_body(xm_full_ref, oh_full_ref, q_ref, ohq_ref, out_ref):
    q = q_ref[...]
    ohq = ohq_ref[...]
    m = jnp.zeros((_BLK, 1), jnp.float32)
    l = jnp.zeros((_BLK, 1), jnp.float32)
    zc = jnp.zeros((_BLK, 1), jnp.float32)
    acc = jnp.zeros((_BLK, _D), jnp.float32)
    scale = 1.0 / (_D ** 0.5)
    for j in range(_N // _BLK):
        k = xm_full_ref[pl.ds(j * _BLK, _BLK), :]
        ohk = oh_full_ref[pl.ds(j * _BLK, _BLK), :]
        msk = lax.dot_general(ohq, ohk, (((1,), (1,)), ((), ())),
                              preferred_element_type=jnp.float32)
        s = lax.dot_general(q, k, (((1,), (1,)), ((), ())),
                            preferred_element_type=jnp.float32) * scale
        s = jnp.where(msk > 0.5, s, _NEG)
        m_new = jnp.maximum(m, jnp.max(s, axis=1, keepdims=True))
        alpha = jnp.exp(m - m_new)
        p = jnp.exp(s - m_new)
        l = l * alpha + jnp.sum(p, axis=1, keepdims=True)
        acc = acc * alpha + jnp.dot(p, k, preferred_element_type=jnp.float32)
        zc = zc + (float(_BLK) - jnp.sum(msk, axis=1, keepdims=True))
        m = m_new
    den = l + zc * jnp.exp(-m)
    out_ref[...] = acc / den


@jax.jit
def kernel(x, input_mask, rotations):
    b, n, d = x.shape
    x2 = x[0]
    mask2 = input_mask[0][:, None]
    rot = rotations[0, :, -1, :]
    rot_full = jnp.concatenate([rot, -rot], axis=-1)

    xm, oh = pl.pallas_call(
        _prep_body,
        out_shape=(
            jax.ShapeDtypeStruct((_N, _D), jnp.float32),
            jax.ShapeDtypeStruct((_N, _NB), jnp.float32),
        ),
    )(x2, mask2, rot_full)

    nblk = _N // _BLK
    out = pl.pallas_call(
        _attn_body,
        grid=(nblk,),
        in_specs=[
            pl.BlockSpec((_N, _D), lambda i: (0, 0)),
            pl.BlockSpec((_N, _NB), lambda i: (0, 0)),
            pl.BlockSpec((_BLK, _D), lambda i: (i, 0)),
            pl.BlockSpec((_BLK, _NB), lambda i: (i, 0)),
        ],
        out_specs=pl.BlockSpec((_BLK, _D), lambda i: (i, 0)),
        out_shape=jax.ShapeDtypeStruct((_N, _D), jnp.float32),
    )(xm, oh, xm, oh)

    return out[None]


# speedup vs baseline: 5.2992x; 1.0324x over previous
import functools

import jax
import jax.numpy as jnp
from jax import lax
from jax.experimental import pallas as pl

_N = 2048
_D = 768
_NB = 8
_BLK = 256
_EPS = 1e-5
_NEG = -1e30


def _prep_body(x_ref, mask_ref, rot_ref, xm_ref, oh_ref):
    x = x_ref[...]
    mu = jnp.mean(x, axis=1, keepdims=True)
    var = jnp.mean((x - mu) ** 2, axis=1, keepdims=True)
    xn = (x - mu) * lax.rsqrt(var + _EPS)
    s = jnp.dot(xn, rot_ref[...], preferred_element_type=jnp.float32)
    smax = jnp.max(s, axis=1, keepdims=True)
    idx = lax.broadcasted_iota(jnp.int32, s.shape, 1)
    cand = jnp.where(s == smax, idx, _NB)
    first = jnp.min(cand, axis=1, keepdims=True)
    oh_ref[...] = (idx == first).astype(jnp.bfloat16)
    xm_ref[...] = (xn * mask_ref[...]).astype(jnp.bfloat16)


def _attn_body(xm_full_ref, oh_full_ref, q_ref, ohq_ref, out_ref):
    q = q_ref[...]
    ohq = ohq_ref[...]
    m = jnp.zeros((_BLK, 1), jnp.float32)
    l = jnp.zeros((_BLK, 1), jnp.float32)
    zc = jnp.zeros((_BLK, 1), jnp.float32)
    acc = jnp.zeros((_BLK, _D), jnp.float32)
    scale = 1.0 / (_D ** 0.5)
    for j in range(_N // _BLK):
        k = xm_full_ref[pl.ds(j * _BLK, _BLK), :]
        ohk = oh_full_ref[pl.ds(j * _BLK, _BLK), :]
        msk = lax.dot_general(ohq, ohk, (((1,), (1,)), ((), ())),
                              preferred_element_type=jnp.float32)
        s = lax.dot_general(q, k, (((1,), (1,)), ((), ())),
                            preferred_element_type=jnp.float32) * scale
        s = jnp.where(msk > 0.5, s, _NEG)
        m_new = jnp.maximum(m, jnp.max(s, axis=1, keepdims=True))
        alpha = jnp.exp(m - m_new)
        p = jnp.exp(s - m_new)
        l = l * alpha + jnp.sum(p, axis=1, keepdims=True)
        acc = acc * alpha + jnp.dot(p.astype(jnp.bfloat16), k,
                                    preferred_element_type=jnp.float32)
        zc = zc + (float(_BLK) - jnp.sum(msk, axis=1, keepdims=True))
        m = m_new
    den = l + zc * jnp.exp(-m)
    out_ref[...] = acc / den


@jax.jit
def kernel(x, input_mask, rotations):
    b, n, d = x.shape
    x2 = x[0]
    mask2 = input_mask[0][:, None]
    rot = rotations[0, :, -1, :]
    rot_full = jnp.concatenate([rot, -rot], axis=-1)

    xm, oh = pl.pallas_call(
        _prep_body,
        out_shape=(
            jax.ShapeDtypeStruct((_N, _D), jnp.bfloat16),
            jax.ShapeDtypeStruct((_N, _NB), jnp.bfloat16),
        ),
    )(x2, mask2, rot_full)

    nblk = _N // _BLK
    out = pl.pallas_call(
        _attn_body,
        grid=(nblk,),
        in_specs=[
            pl.BlockSpec((_N, _D), lambda i: (0, 0)),
            pl.BlockSpec((_N, _NB), lambda i: (0, 0)),
            pl.BlockSpec((_BLK, _D), lambda i: (i, 0)),
            pl.BlockSpec((_BLK, _NB), lambda i: (i, 0)),
        ],
        out_specs=pl.BlockSpec((_BLK, _D), lambda i: (i, 0)),
        out_shape=jax.ShapeDtypeStruct((_N, _D), jnp.float32),
    )(xm, oh, xm, oh)

    return out[None]


# speedup vs baseline: 7.7054x; 1.4541x over previous
import jax
import jax.numpy as jnp
from jax import lax
from jax.experimental import pallas as pl

_N = 2048
_D = 768
_NB = 8
_BLK = 256
_AUG = 128
_DA = _D + _AUG
_EPS = 1e-5
_IND4 = 1.0 / (_D ** 0.25)
_D4 = _D ** 0.25
_SHIFT = 64.0 + 1.0 + 28.0
_MU = 28.0


def _prep_body(x_ref, mask_ref, rot_ref, xa_ref, z_ref):
    x = x_ref[...]
    mu = jnp.mean(x, axis=1, keepdims=True)
    var = jnp.mean((x - mu) ** 2, axis=1, keepdims=True)
    xn = (x - mu) * lax.rsqrt(var + _EPS)
    xm = xn * mask_ref[...]
    s = jnp.dot(xn, rot_ref[...], preferred_element_type=jnp.float32)
    smax = jnp.max(s, axis=1, keepdims=True)
    idx8 = lax.broadcasted_iota(jnp.int32, s.shape, 1)
    cand = jnp.where(s == smax, idx8, _NB)
    first = jnp.min(cand, axis=1, keepdims=True)
    oh = (idx8 == first).astype(jnp.float32)
    cnt = jnp.sum(oh, axis=0, keepdims=True)
    z_ref[...] = float(_N) - lax.dot_general(
        oh, cnt, (((1,), (1,)), ((), ())),
        preferred_element_type=jnp.float32)
    idx128 = lax.broadcasted_iota(jnp.int32, (_N, _AUG), 1)
    aug = jnp.where(idx128 == first, 8.0, 0.0) + jnp.where(idx128 == _NB, 1.0, 0.0)
    xa_ref[...] = jnp.concatenate(
        [xm * _IND4, aug], axis=1).astype(jnp.bfloat16)


def _attn_body(xa_full_ref, qa_ref, z_ref, out_ref):
    qa = qa_ref[...]
    xa = xa_full_ref[...]
    s = lax.dot_general(qa, xa, (((1,), (1,)), ((), ())),
                        preferred_element_type=jnp.float32)
    p = jnp.exp(s - _SHIFT)
    acc = lax.dot_general(p.astype(jnp.bfloat16), xa,
                          (((1,), (0,)), ((), ())),
                          preferred_element_type=jnp.float32)
    l = acc[:, _D + _NB:_D + _NB + 1]
    den = l + z_ref[...] * jnp.exp(-_MU)
    out_ref[...] = acc[:, :_D] * (_D4 / den)


@jax.jit
def kernel(x, input_mask, rotations):
    x2 = x[0]
    mask2 = input_mask[0][:, None]
    rot = rotations[0, :, -1, :]
    rot_full = jnp.concatenate([rot, -rot], axis=-1)

    xa, z = pl.pallas_call(
        _prep_body,
        out_shape=(
            jax.ShapeDtypeStruct((_N, _DA), jnp.bfloat16),
            jax.ShapeDtypeStruct((_N, 1), jnp.float32),
        ),
    )(x2, mask2, rot_full)

    nblk = _N // _BLK
    out = pl.pallas_call(
        _attn_body,
        grid=(nblk,),
        in_specs=[
            pl.BlockSpec((_N, _DA), lambda i: (0, 0)),
            pl.BlockSpec((_BLK, _DA), lambda i: (i, 0)),
            pl.BlockSpec((_BLK, 1), lambda i: (i, 0)),
        ],
        out_specs=pl.BlockSpec((_BLK, _D), lambda i: (i, 0)),
        out_shape=jax.ShapeDtypeStruct((_N, _D), jnp.float32),
    )(xa, xa, z)

    return out[None]


# speedup vs baseline: 8.0432x; 1.0438x over previous
import jax
import jax.numpy as jnp
from jax import lax
from jax.experimental import pallas as pl

_N = 2048
_D = 768
_NB = 8
_BLK = 512
_AUG = 128
_DA = _D + _AUG
_EPS = 1e-5
_IND4 = 1.0 / (_D ** 0.25)
_D4 = _D ** 0.25
_SHIFT = 64.0 + 1.0 + 28.0
_MU = 28.0


def _prep_body(x_ref, mask_ref, rot_ref, xa_ref, z_ref):
    x = x_ref[...]
    mu = jnp.mean(x, axis=1, keepdims=True)
    var = jnp.mean((x - mu) ** 2, axis=1, keepdims=True)
    xn = (x - mu) * lax.rsqrt(var + _EPS)
    xm = xn * mask_ref[...]
    s = jnp.dot(xn, rot_ref[...], preferred_element_type=jnp.float32)
    smax = jnp.max(s, axis=1, keepdims=True)
    idx8 = lax.broadcasted_iota(jnp.int32, s.shape, 1)
    cand = jnp.where(s == smax, idx8, _NB)
    first = jnp.min(cand, axis=1, keepdims=True)
    oh = (idx8 == first).astype(jnp.float32)
    cnt = jnp.sum(oh, axis=0, keepdims=True)
    z_ref[...] = float(_N) - lax.dot_general(
        oh, cnt, (((1,), (1,)), ((), ())),
        preferred_element_type=jnp.float32)
    idx128 = lax.broadcasted_iota(jnp.int32, (_N, _AUG), 1)
    aug = jnp.where(idx128 == first, 8.0, 0.0) + jnp.where(idx128 == _NB, 1.0, 0.0)
    xa_ref[...] = jnp.concatenate(
        [xm * _IND4, aug], axis=1).astype(jnp.bfloat16)


def _attn_body(xa_full_ref, qa_ref, z_ref, out_ref):
    qa = qa_ref[...]
    xa = xa_full_ref[...]
    s = lax.dot_general(qa, xa, (((1,), (1,)), ((), ())),
                        preferred_element_type=jnp.float32)
    p = jnp.exp(s - _SHIFT)
    acc = lax.dot_general(p.astype(jnp.bfloat16), xa,
                          (((1,), (0,)), ((), ())),
                          preferred_element_type=jnp.float32)
    l = acc[:, _D + _NB:_D + _NB + 1]
    den = l + z_ref[...] * jnp.exp(-_MU)
    out_ref[...] = acc[:, :_D] * (_D4 / den)


@jax.jit
def kernel(x, input_mask, rotations):
    x2 = x[0]
    mask2 = input_mask[0][:, None]
    rot = rotations[0, :, -1, :]
    rot_full = jnp.concatenate([rot, -rot], axis=-1)

    xa, z = pl.pallas_call(
        _prep_body,
        out_shape=(
            jax.ShapeDtypeStruct((_N, _DA), jnp.bfloat16),
            jax.ShapeDtypeStruct((_N, 1), jnp.float32),
        ),
    )(x2, mask2, rot_full)

    nblk = _N // _BLK
    out = pl.pallas_call(
        _attn_body,
        grid=(nblk,),
        in_specs=[
            pl.BlockSpec((_N, _DA), lambda i: (0, 0)),
            pl.BlockSpec((_BLK, _DA), lambda i: (i, 0)),
            pl.BlockSpec((_BLK, 1), lambda i: (i, 0)),
        ],
        out_specs=pl.BlockSpec((_BLK, _D), lambda i: (i, 0)),
        out_shape=jax.ShapeDtypeStruct((_N, _D), jnp.float32),
    )(xa, xa, z)

    return out[None]


# speedup vs baseline: 9.2160x; 1.1458x over previous
import jax
import jax.numpy as jnp
from jax import lax
from jax.experimental import pallas as pl
from jax.experimental.pallas import tpu as pltpu

_N = 2048
_D = 768
_NB = 8
_BLK = 512
_AUG = 128
_DA = _D + _AUG
_EPS = 1e-5
_IND4 = 1.0 / (_D ** 0.25)
_D4 = _D ** 0.25
_SHIFT = 64.0 + 1.0 + 28.0
_MU = 28.0


def _body(x_ref, mask_ref, rot_ref, out_ref, xa_s, z_s):
    i = pl.program_id(0)

    @pl.when(i == 0)
    def _prep():
        x = x_ref[...]
        mu = jnp.mean(x, axis=1, keepdims=True)
        var = jnp.mean((x - mu) ** 2, axis=1, keepdims=True)
        xn = (x - mu) * lax.rsqrt(var + _EPS)
        xm = xn * mask_ref[...]
        rot = rot_ref[...]
        s4 = jnp.dot(xn, rot, preferred_element_type=jnp.float32)
        s = jnp.concatenate([s4, -s4], axis=1)
        smax = jnp.max(s, axis=1, keepdims=True)
        idx8 = lax.broadcasted_iota(jnp.int32, s.shape, 1)
        cand = jnp.where(s == smax, idx8, _NB)
        first = jnp.min(cand, axis=1, keepdims=True)
        oh = (idx8 == first).astype(jnp.float32)
        cnt = jnp.sum(oh, axis=0, keepdims=True)
        z_s[...] = float(_N) - lax.dot_general(
            oh, cnt, (((1,), (1,)), ((), ())),
            preferred_element_type=jnp.float32)
        idx128 = lax.broadcasted_iota(jnp.int32, (_N, _AUG), 1)
        aug = (jnp.where(idx128 == first, 8.0, 0.0)
               + jnp.where(idx128 == _NB, 1.0, 0.0))
        xa_s[...] = jnp.concatenate(
            [xm * _IND4, aug], axis=1).astype(jnp.bfloat16)

    @pl.when(i > 0)
    def _attn():
        r0 = (i - 1) * _BLK
        qa = xa_s[pl.ds(r0, _BLK), :]
        xa = xa_s[...]
        s = lax.dot_general(qa, xa, (((1,), (1,)), ((), ())),
                            preferred_element_type=jnp.float32)
        p = jnp.exp(s - _SHIFT)
        acc = lax.dot_general(p.astype(jnp.bfloat16), xa,
                              (((1,), (0,)), ((), ())),
                              preferred_element_type=jnp.float32)
        l = acc[:, _D + _NB:_D + _NB + 1]
        den = l + z_s[pl.ds(r0, _BLK), :] * jnp.exp(-_MU)
        out_ref[...] = acc[:, :_D] * (_D4 / den)


@jax.jit
def kernel(x, input_mask, rotations):
    x2 = x[0]
    mask2 = input_mask[0][:, None]
    rot = rotations[0, :, -1, :]
    nblk = _N // _BLK
    out = pl.pallas_call(
        _body,
        grid=(nblk + 1,),
        in_specs=[
            pl.BlockSpec((_N, _D), lambda i: (0, 0)),
            pl.BlockSpec((_N, 1), lambda i: (0, 0)),
            pl.BlockSpec((_D, _NB // 2), lambda i: (0, 0)),
        ],
        out_specs=pl.BlockSpec((_BLK, _D),
                               lambda i: (jnp.maximum(i - 1, 0), 0)),
        out_shape=jax.ShapeDtypeStruct((_N, _D), jnp.float32),
        scratch_shapes=[
            pltpu.VMEM((_N, _DA), jnp.bfloat16),
            pltpu.VMEM((_N, 1), jnp.float32),
        ],
    )(x2, mask2, rot)

    return out[None]


# speedup vs baseline: 9.3795x; 1.0177x over previous
import jax
import jax.numpy as jnp
from jax import lax
from jax.experimental import pallas as pl
from jax.experimental.pallas import tpu as pltpu

_N = 2048
_D = 768
_NB = 8
_BLK = 1024
_AUG = 128
_DA = _D + _AUG
_EPS = 1e-5
_IND4 = 1.0 / (_D ** 0.25)
_D4 = _D ** 0.25
_SHIFT = 64.0 + 1.0 + 28.0
_MU = 28.0


def _body(x_ref, mask_ref, rot_ref, out_ref, xa_s, z_s):
    i = pl.program_id(0)

    @pl.when(i == 0)
    def _prep():
        x = x_ref[...]
        mu = jnp.mean(x, axis=1, keepdims=True)
        var = jnp.mean((x - mu) ** 2, axis=1, keepdims=True)
        xn = (x - mu) * lax.rsqrt(var + _EPS)
        xm = xn * mask_ref[...]
        rot = rot_ref[...]
        s4 = jnp.dot(xn, rot, preferred_element_type=jnp.float32)
        s = jnp.concatenate([s4, -s4], axis=1)
        smax = jnp.max(s, axis=1, keepdims=True)
        idx8 = lax.broadcasted_iota(jnp.int32, s.shape, 1)
        cand = jnp.where(s == smax, idx8, _NB)
        first = jnp.min(cand, axis=1, keepdims=True)
        oh = (idx8 == first).astype(jnp.float32)
        cnt = jnp.sum(oh, axis=0, keepdims=True)
        z_s[...] = float(_N) - lax.dot_general(
            oh, cnt, (((1,), (1,)), ((), ())),
            preferred_element_type=jnp.float32)
        idx128 = lax.broadcasted_iota(jnp.int32, (_N, _AUG), 1)
        aug = (jnp.where(idx128 == first, 8.0, 0.0)
               + jnp.where(idx128 == _NB, 1.0, 0.0))
        xa_s[...] = jnp.concatenate(
            [xm * _IND4, aug], axis=1).astype(jnp.bfloat16)

    @pl.when(i > 0)
    def _attn():
        r0 = (i - 1) * _BLK
        qa = xa_s[pl.ds(r0, _BLK), :]
        xa = xa_s[...]
        s = lax.dot_general(qa, xa, (((1,), (1,)), ((), ())),
                            preferred_element_type=jnp.float32)
        p = jnp.exp(s - _SHIFT)
        acc = lax.dot_general(p.astype(jnp.bfloat16), xa,
                              (((1,), (0,)), ((), ())),
                              preferred_element_type=jnp.float32)
        l = acc[:, _D + _NB:_D + _NB + 1]
        den = l + z_s[pl.ds(r0, _BLK), :] * jnp.exp(-_MU)
        out_ref[...] = acc[:, :_D] * (_D4 / den)


@jax.jit
def kernel(x, input_mask, rotations):
    x2 = x[0]
    mask2 = input_mask[0][:, None]
    rot = rotations[0, :, -1, :]
    nblk = _N // _BLK
    out = pl.pallas_call(
        _body,
        grid=(nblk + 1,),
        in_specs=[
            pl.BlockSpec((_N, _D), lambda i: (0, 0)),
            pl.BlockSpec((_N, 1), lambda i: (0, 0)),
            pl.BlockSpec((_D, _NB // 2), lambda i: (0, 0)),
        ],
        out_specs=pl.BlockSpec((_BLK, _D),
                               lambda i: (jnp.maximum(i - 1, 0), 0)),
        out_shape=jax.ShapeDtypeStruct((_N, _D), jnp.float32),
        scratch_shapes=[
            pltpu.VMEM((_N, _DA), jnp.bfloat16),
            pltpu.VMEM((_N, 1), jnp.float32),
        ],
    )(x2, mask2, rot)

    return out[None]
